# baseline (speedup 1.0000x reference)
import jax
import jax.numpy as jnp
from jax import lax
from jax.experimental import pallas as pl
from jax.experimental.pallas import tpu as pltpu

_CHUNK = 1024
_DEPTH = 4


def _body(x_hbm, w1_ref, thr_ref, w2_ref, o_ref, x_buf, sems):
    n_chunks = x_hbm.shape[0] // _CHUNK

    def copy(i, slot):
        return pltpu.make_async_copy(
            x_hbm.at[pl.ds(i * _CHUNK, _CHUNK), :],
            x_buf.at[slot],
            sems.at[slot],
        )

    for j in range(_DEPTH):
        copy(j, j).start()

    w1 = w1_ref[:]
    thr = thr_ref[:]
    w2 = w2_ref[:]
    for i in range(n_chunks):
        slot = i % _DEPTH
        copy(i, slot).wait()
        u = jnp.dot(x_buf[slot], w1, preferred_element_type=jnp.float32)
        h = jnp.tanh(u - thr)
        o_ref[pl.ds(i * _CHUNK, _CHUNK), :] = jnp.tanh(
            jnp.dot(h, w2, preferred_element_type=jnp.float32) - 0.5
        )
        if i + _DEPTH < n_chunks:
            copy(i + _DEPTH, slot).start()


def kernel(x, W1, thr_h, W2):
    batch, in_size = x.shape
    hidden = W1.shape[1]
    out_size = W2.shape[1]
    thr2d = thr_h.reshape(1, hidden)

    return pl.pallas_call(
        _body,
        in_specs=[
            pl.BlockSpec(memory_space=pl.ANY),
            pl.BlockSpec(memory_space=pltpu.VMEM),
            pl.BlockSpec(memory_space=pltpu.VMEM),
            pl.BlockSpec(memory_space=pltpu.VMEM),
        ],
        out_specs=pl.BlockSpec(memory_space=pltpu.VMEM),
        out_shape=jax.ShapeDtypeStruct((batch, out_size), jnp.float32),
        scratch_shapes=[
            pltpu.VMEM((_DEPTH, _CHUNK, in_size), jnp.float32),
            pltpu.SemaphoreType.DMA((_DEPTH,)),
        ],
    )(x, W1, thr2d, W2)
